# Initial kernel scaffold; baseline (speedup 1.0000x reference)
import jax
import jax.numpy as jnp
from jax.experimental import pallas as pl


def kernel(logits):
    raise NotImplementedError("write your pallas kernel here")


# speedup vs baseline: 86.4788x; 86.4788x over previous
import jax
import jax.numpy as jnp
from jax.experimental import pallas as pl
from jax.experimental.pallas import tpu as pltpu

_T = 1.3
_TAU = 0.9
_KREG = 5
_LAMDA = 0.01

_ROWS = 8


def _body(x_ref, o_ref):
    R, V = x_ref.shape
    y = x_ref[...] / _T
    m = jnp.max(y, axis=1, keepdims=True)
    e = jnp.exp(y - m)
    o_ref[...] = e
    Z = jnp.sum(e, axis=1, keepdims=True)

    hi0 = jnp.full((R, 1), 0x3FC00000, jnp.int32)
    lo0 = jnp.zeros((R, 1), jnp.int32)

    def bisect(_, carry):
        lo, hi = carry
        mid = (lo + hi) // 2
        t = jax.lax.bitcast_convert_type(mid, jnp.float32)
        ev = o_ref[...]
        sel = ev >= t
        s = jnp.sum(jnp.where(sel, ev, 0.0), axis=1, keepdims=True)
        cnt = jnp.sum(jnp.where(sel, 1.0, 0.0), axis=1, keepdims=True)
        G = s / Z + _LAMDA * jnp.maximum(0.0, cnt - _KREG)
        gt = G > _TAU
        return jnp.where(gt, mid, lo), jnp.where(gt, hi, mid)

    lo, hi = jax.lax.fori_loop(0, 31, bisect, (lo0, hi0))
    vc = jax.lax.bitcast_convert_type(lo, jnp.float32)

    ev = o_ref[...]
    gtm = ev > vc
    eqm = ev == vc
    n_prev = jnp.sum(jnp.where(gtm, 1.0, 0.0), axis=1, keepdims=True)
    s_prev = jnp.sum(jnp.where(gtm, ev, 0.0), axis=1, keepdims=True)
    m_eq = jnp.sum(jnp.where(eqm, 1.0, 0.0), axis=1, keepdims=True)

    r = jax.lax.broadcasted_iota(jnp.int32, (R, 128), 1).astype(jnp.float32) + 1.0
    f_r = (s_prev + r * vc) / Z + _LAMDA * jnp.maximum(0.0, n_prev + r - _KREG)
    ok = jnp.where((f_r <= _TAU) & (r <= m_eq), 1.0, 0.0)
    e_needed = jnp.sum(ok, axis=1, keepdims=True) + 1.0

    idx = jax.lax.broadcasted_iota(jnp.int32, (R, V), 1)
    plo0 = jnp.zeros((R, 1), jnp.int32)
    phi0 = jnp.full((R, 1), V, jnp.int32)

    def psearch(_, carry):
        plo, phi = carry
        pmid = (plo + phi) // 2
        cnt = jnp.sum(jnp.where(eqm & (idx < pmid), 1.0, 0.0), axis=1, keepdims=True)
        ge = cnt >= e_needed
        return jnp.where(ge, plo, pmid), jnp.where(ge, pmid, phi)

    _, p = jax.lax.fori_loop(0, 17, psearch, (plo0, phi0))

    mask = gtm | (eqm & (idx < p))
    o_ref[...] = jnp.where(mask, ev * (1.0 / Z), 0.0)


def kernel(logits):
    B, V = logits.shape
    grid = (B // _ROWS,)
    return pl.pallas_call(
        _body,
        grid=grid,
        in_specs=[pl.BlockSpec((_ROWS, V), lambda i: (i, 0))],
        out_specs=pl.BlockSpec((_ROWS, V), lambda i: (i, 0)),
        out_shape=jax.ShapeDtypeStruct((B, V), jnp.float32),
    )(logits)
